# baseline (speedup 1.0000x reference)
import functools

import jax
import jax.numpy as jnp
from jax import lax
from jax.experimental import pallas as pl
from jax.experimental.pallas import tpu as pltpu
from jax.experimental.pallas import tpu_sc as plsc

N = 10000
E = 320000
D_IN = 128
D_HID = 16

NC = 2
NS = 16
NW = NC * NS

N_PAD = 10240
TRASH = N
CHUNK = 128
NCHUNK = 80
EPT = NCHUNK * CHUNK
E_PAD = NW * EPT
ROWS_PER_TILE = N_PAD // NS
TOT_CHUNKS = E_PAD // CHUNK
K_C0 = 110
K_C1 = 50
C1_BASE = NS * K_C0
EPT_C0 = 12032
EPT_C1 = 8448
E_C1_BASE = NS * EPT_C0

_mesh = plsc.VectorSubcoreMesh(core_axis_name="c", subcore_axis_name="s")
_sc_params = pltpu.CompilerParams(needs_layout_passes=False)
_sc_params_sctile = pltpu.CompilerParams(
    needs_layout_passes=False, use_tc_tiling_on_sc=False
)


def _zero_vmem(ref, n):
    zv = jnp.zeros((16,), jnp.float32)

    def zb(i, carry):
        ref[pl.ds(i * 16, 16)] = zv
        return carry

    lax.fori_loop(0, n // 16, zb, 0)


@functools.partial(
    pl.kernel,
    out_type=jax.ShapeDtypeStruct((NW, N_PAD), jnp.float32),
    mesh=_mesh,
    scratch_types=[
        pltpu.VMEM((EPT_C0,), jnp.int32),
        pltpu.VMEM((EPT_C0,), jnp.int32),
        pltpu.VMEM((N_PAD,), jnp.float32),
        pltpu.VMEM((N_PAD,), jnp.float32),
    ],
    compiler_params=_sc_params,
)
def _sc_agg_w1(table_hbm, src_hbm, dst_hbm, out_hbm,
               src_v, dst_v, tab_v, acc_v):
    c = lax.axis_index("c")
    s = lax.axis_index("s")
    wid = s * NC + c
    pltpu.sync_copy(table_hbm, tab_v)
    _zero_vmem(acc_v, N_PAD)

    def run(ept, base):
        pltpu.sync_copy(src_hbm.at[pl.ds(base, ept)], src_v.at[pl.ds(0, ept)])
        pltpu.sync_copy(dst_hbm.at[pl.ds(base, ept)], dst_v.at[pl.ds(0, ept)])

        def body(i, carry):
            for u in range(4):
                srci = src_v[pl.ds((i * 4 + u) * 16, 16)]
                dsti = dst_v[pl.ds((i * 4 + u) * 16, 16)]
                vals = plsc.load_gather(tab_v, [srci])
                plsc.addupdate_scatter(acc_v, [dsti], vals)
            return carry

        lax.fori_loop(0, ept // 64, body, 0)

    @pl.when(c == 0)
    def _():
        run(EPT_C0, s * EPT_C0)

    @pl.when(c == 1)
    def _():
        run(EPT_C1, E_C1_BASE + s * EPT_C1)

    pltpu.sync_copy(acc_v, out_hbm.at[wid])


@functools.partial(
    pl.kernel,
    out_type=jax.ShapeDtypeStruct((NW, N_PAD), jnp.float32),
    mesh=_mesh,
    scratch_types=[
        pltpu.VMEM((EPT_C0,), jnp.int32),
        pltpu.VMEM((N_PAD,), jnp.float32),
    ],
    compiler_params=_sc_params,
)
def _sc_deg(dst_hbm, out_hbm, dst_v, acc_v):
    c = lax.axis_index("c")
    s = lax.axis_index("s")
    wid = s * NC + c
    _zero_vmem(acc_v, N_PAD)
    ones = jnp.ones((16,), jnp.float32)

    def run(ept, base):
        pltpu.sync_copy(dst_hbm.at[pl.ds(base, ept)], dst_v.at[pl.ds(0, ept)])

        def body(i, carry):
            for u in range(4):
                dsti = dst_v[pl.ds((i * 4 + u) * 16, 16)]
                plsc.addupdate_scatter(acc_v, [dsti], ones)
            return carry

        lax.fori_loop(0, ept // 64, body, 0)

    @pl.when(c == 0)
    def _():
        run(EPT_C0, s * EPT_C0)

    @pl.when(c == 1)
    def _():
        run(EPT_C1, E_C1_BASE + s * EPT_C1)

    pltpu.sync_copy(acc_v, out_hbm.at[wid])


@functools.partial(
    pl.kernel,
    out_type=jax.ShapeDtypeStruct((NC * N_PAD, D_HID), jnp.float32),
    mesh=_mesh,
    scratch_types=[
        pltpu.VMEM((K_C0, CHUNK), jnp.int32),
        pltpu.VMEM((K_C0, CHUNK), jnp.int32),
        pltpu.VMEM((6, CHUNK, D_HID), jnp.float32),
        pltpu.VMEM((ROWS_PER_TILE, D_HID), jnp.float32),
        pltpu.VMEM((5, CHUNK), jnp.int32),
        pltpu.VMEM_SHARED((N_PAD, D_HID), jnp.float32),
    ] + [pltpu.SemaphoreType.DMA] * 12,
    compiler_params=_sc_params_sctile,
)
def _sc_agg_w16(table_hbm, src_hbm, dst_hbm, out_hbm,
                src_v, dst_v, rows_v, obuf_v, oidx_v, acc_sh, *sems):
    NB = 6
    AH = 3
    gsem = sems[:NB]
    ssem = sems[NB:]
    c = lax.axis_index("c")
    s = lax.axis_index("s")
    wid = s * NC + c
    r0 = s * ROWS_PER_TILE
    zv = jnp.zeros((D_HID,), jnp.float32)

    def zbody(r, carry):
        rows_v[0, r] = zv
        return carry

    lax.fori_loop(0, CHUNK, zbody, 0)
    for t in range(ROWS_PER_TILE // CHUNK):
        pltpu.sync_copy(rows_v.at[0],
                        acc_sh.at[pl.ds(r0 + t * CHUNK, CHUNK)])
    plsc.subcore_barrier()

    def issue_gather(j, b):
        pltpu.async_copy(table_hbm.at[src_v.at[j]], rows_v.at[b], gsem[b])

    def wait_gather(j, b):
        pltpu.make_async_copy(
            table_hbm.at[src_v.at[j]], rows_v.at[b], gsem[b]
        ).wait()

    def issue_scatter(j, b):
        pltpu.async_copy(
            rows_v.at[b], acc_sh.at[dst_v.at[j]], ssem[b], add=True
        )

    def wait_scatter(j, b):
        pltpu.make_async_copy(
            rows_v.at[b], acc_sh.at[dst_v.at[j]], ssem[b]
        ).wait()

    def pipe(nchunk, start):
        pltpu.sync_copy(src_hbm.at[pl.ds(start, nchunk)],
                        src_v.at[pl.ds(0, nchunk)])
        pltpu.sync_copy(dst_hbm.at[pl.ds(start, nchunk)],
                        dst_v.at[pl.ds(0, nchunk)])

        for j in range(AH):
            issue_gather(j, j)
        for j in range(AH):
            wait_gather(j, j)
            issue_scatter(j, j)
            issue_gather(j + AH, j + AH)

        def body(j2, carry):
            for u in range(NB):
                j = AH + j2 * NB + u
                b = (AH + u) % NB
                wait_gather(j, b)
                issue_scatter(j, b)
                bw = (AH + u + AH) % NB
                wait_scatter(j - AH, bw)
                issue_gather(j + AH, bw)
            return carry

        steady = nchunk - AH - (NB - 1)
        assert steady % NB == 0
        lax.fori_loop(0, steady // NB, body, 0)

        for j in range(AH + steady, nchunk):
            b = j % NB
            wait_gather(j, b)
            issue_scatter(j, b)
            if j + AH < nchunk:
                bw = (j + AH) % NB
                wait_scatter(j - AH, bw)
                issue_gather(j + AH, bw)
        for j in range(nchunk - NB, nchunk):
            wait_scatter(j, j % NB)

    @pl.when(c == 0)
    def _():
        pipe(K_C0, s * K_C0)

    @pl.when(c == 1)
    def _():
        pipe(K_C1, C1_BASE + s * K_C1)

    plsc.subcore_barrier()

    obase = c * N_PAD + r0
    iota16 = lax.iota(jnp.int32, 16)
    for t in range(ROWS_PER_TILE // CHUNK):
        for u in range(CHUNK // 16):
            oidx_v[t, pl.ds(u * 16, 16)] = obase + t * CHUNK + u * 16 + iota16
    pltpu.sync_copy(acc_sh.at[pl.ds(r0, ROWS_PER_TILE)], obuf_v)
    for t in range(ROWS_PER_TILE // CHUNK):
        pltpu.async_copy(
            obuf_v.at[pl.ds(t * CHUNK, CHUNK)],
            out_hbm.at[oidx_v.at[t]],
            gsem[t % NB],
        )
    for t in range(ROWS_PER_TILE // CHUNK):
        pltpu.make_async_copy(
            obuf_v.at[pl.ds(t * CHUNK, CHUNK)],
            out_hbm.at[oidx_v.at[t]],
            gsem[t % NB],
        ).wait()


BLK = 2048
GRID = N_PAD // BLK


def _tc1_body(x_ref, w_ref, dp_ref, h1s_ref, dis_ref):
    deg = jnp.sum(dp_ref[...], axis=0) + 1.0
    dis = lax.rsqrt(deg)[:, None]
    h = jnp.dot(x_ref[...], w_ref[...], preferred_element_type=jnp.float32)
    h1s_ref[...] = h * dis
    dis_ref[...] = jnp.broadcast_to(dis, (BLK, D_HID))


_tc1 = pl.pallas_call(
    _tc1_body,
    grid=(GRID,),
    in_specs=[
        pl.BlockSpec((BLK, D_IN), lambda i: (i, 0)),
        pl.BlockSpec((D_IN, D_HID), lambda i: (0, 0)),
        pl.BlockSpec((NW, BLK), lambda i: (0, i)),
    ],
    out_specs=[
        pl.BlockSpec((BLK, D_HID), lambda i: (i, 0)),
        pl.BlockSpec((BLK, D_HID), lambda i: (i, 0)),
    ],
    out_shape=[
        jax.ShapeDtypeStruct((N_PAD, D_HID), jnp.float32),
        jax.ShapeDtypeStruct((N_PAD, D_HID), jnp.float32),
    ],
)


def _tc2_body(acc_ref, h1s_ref, dis_ref, b1_ref, w2_ref, h2s_ref):
    tot = acc_ref[0] + acc_ref[1] + h1s_ref[...]
    out1 = tot * dis_ref[...] + b1_ref[...]
    a1 = jnp.maximum(out1, 0.0)
    h2 = jnp.dot(a1, w2_ref[...], preferred_element_type=jnp.float32)
    h2s_ref[...] = h2 * dis_ref[:, :1]


_tc2 = pl.pallas_call(
    _tc2_body,
    grid=(GRID,),
    in_specs=[
        pl.BlockSpec((NC, BLK, D_HID), lambda i: (0, i, 0)),
        pl.BlockSpec((BLK, D_HID), lambda i: (i, 0)),
        pl.BlockSpec((BLK, D_HID), lambda i: (i, 0)),
        pl.BlockSpec((1, D_HID), lambda i: (0, 0)),
        pl.BlockSpec((D_HID, 1), lambda i: (0, 0)),
    ],
    out_specs=pl.BlockSpec((BLK, 1), lambda i: (i, 0)),
    out_shape=jax.ShapeDtypeStruct((N_PAD, 1), jnp.float32),
)


def _tc3_body(a2p_ref, h2s_ref, dis_ref, b2_ref, out_ref):
    tot = jnp.sum(a2p_ref[...], axis=0)[:, None] + h2s_ref[...]
    out_ref[...] = tot * dis_ref[:, :1] + b2_ref[...]


_tc3 = pl.pallas_call(
    _tc3_body,
    grid=(GRID,),
    in_specs=[
        pl.BlockSpec((NW, BLK), lambda i: (0, i)),
        pl.BlockSpec((BLK, 1), lambda i: (i, 0)),
        pl.BlockSpec((BLK, D_HID), lambda i: (i, 0)),
        pl.BlockSpec((1, 1), lambda i: (0, 0)),
    ],
    out_specs=pl.BlockSpec((BLK, 1), lambda i: (i, 0)),
    out_shape=jax.ShapeDtypeStruct((N_PAD, 1), jnp.float32),
)


def kernel(x, edge_index, W1, b1, W2, b2):
    ei = edge_index.astype(jnp.int32)
    pad = jnp.full((E_PAD - E,), TRASH, jnp.int32)
    src = jnp.concatenate([ei[0], pad])
    dst = jnp.concatenate([ei[1], pad])
    src2 = src.reshape(TOT_CHUNKS, CHUNK)
    dst2 = dst.reshape(TOT_CHUNKS, CHUNK)

    x_pad = jnp.pad(x, ((0, N_PAD - N), (0, 0)))
    deg_parts = _sc_deg(dst)
    h1s, dis16 = _tc1(x_pad, W1, deg_parts)
    acc = _sc_agg_w16(h1s, src2, dst2).reshape(NC, N_PAD, D_HID)
    h2s = _tc2(acc, h1s, dis16, b1.reshape(1, D_HID), W2)
    acc2 = _sc_agg_w1(h2s.reshape(-1), src, dst)
    out = _tc3(acc2, h2s, dis16, b2.reshape(1, 1))
    return out[:N]


# speedup vs baseline: 1.0058x; 1.0058x over previous
import functools

import jax
import jax.numpy as jnp
from jax import lax
from jax.experimental import pallas as pl
from jax.experimental.pallas import tpu as pltpu
from jax.experimental.pallas import tpu_sc as plsc

N = 10000
E = 320000
D_IN = 128
D_HID = 16

NC = 2
NS = 16
NW = NC * NS

N_PAD = 10240
TRASH = N
CHUNK = 128
NCHUNK = 80
EPT = NCHUNK * CHUNK
E_PAD = NW * EPT
ROWS_PER_TILE = N_PAD // NS
TOT_CHUNKS = E_PAD // CHUNK
K_C0 = 128
K_C1 = 32
C1_BASE = NS * K_C0
EPT_C0 = 12032
EPT_C1 = 8448
E_C1_BASE = NS * EPT_C0

_mesh = plsc.VectorSubcoreMesh(core_axis_name="c", subcore_axis_name="s")
_sc_params = pltpu.CompilerParams(needs_layout_passes=False)
_sc_params_sctile = pltpu.CompilerParams(
    needs_layout_passes=False, use_tc_tiling_on_sc=False
)


def _zero_vmem(ref, n):
    zv = jnp.zeros((16,), jnp.float32)

    def zb(i, carry):
        ref[pl.ds(i * 16, 16)] = zv
        return carry

    lax.fori_loop(0, n // 16, zb, 0)


def _spmem_tree_reduce(acc_v, sp_sh, red_v, out_v, out_hbm, c, s):
    pltpu.sync_copy(acc_v, sp_sh.at[s])
    plsc.subcore_barrier()
    r0 = s * ROWS_PER_TILE
    for k in range(NS):
        pltpu.sync_copy(sp_sh.at[k].at[pl.ds(r0, ROWS_PER_TILE)], red_v.at[k])

    def rbody(g, carry):
        tot = red_v[0, pl.ds(g * 16, 16)]
        for k in range(1, NS):
            tot = tot + red_v[k, pl.ds(g * 16, 16)]
        out_v[pl.ds(g * 16, 16)] = tot
        return carry

    lax.fori_loop(0, ROWS_PER_TILE // 16, rbody, 0)
    pltpu.sync_copy(out_v, out_hbm.at[c].at[pl.ds(r0, ROWS_PER_TILE)])


@functools.partial(
    pl.kernel,
    out_type=jax.ShapeDtypeStruct((NC, N_PAD), jnp.float32),
    mesh=_mesh,
    scratch_types=[
        pltpu.VMEM((EPT_C0,), jnp.int32),
        pltpu.VMEM((EPT_C0,), jnp.int32),
        pltpu.VMEM((N_PAD,), jnp.float32),
        pltpu.VMEM((N_PAD,), jnp.float32),
        pltpu.VMEM((NS, ROWS_PER_TILE), jnp.float32),
        pltpu.VMEM((ROWS_PER_TILE,), jnp.float32),
        pltpu.VMEM_SHARED((NS, N_PAD), jnp.float32),
    ],
    compiler_params=_sc_params,
)
def _sc_agg_w1(table_hbm, src_hbm, dst_hbm, out_hbm,
               src_v, dst_v, tab_v, acc_v, red_v, out_v, sp_sh):
    c = lax.axis_index("c")
    s = lax.axis_index("s")
    pltpu.sync_copy(table_hbm, tab_v)
    _zero_vmem(acc_v, N_PAD)

    def run(ept, base):
        pltpu.sync_copy(src_hbm.at[pl.ds(base, ept)], src_v.at[pl.ds(0, ept)])
        pltpu.sync_copy(dst_hbm.at[pl.ds(base, ept)], dst_v.at[pl.ds(0, ept)])

        def body(i, carry):
            for u in range(4):
                srci = src_v[pl.ds((i * 4 + u) * 16, 16)]
                dsti = dst_v[pl.ds((i * 4 + u) * 16, 16)]
                vals = plsc.load_gather(tab_v, [srci])
                plsc.addupdate_scatter(acc_v, [dsti], vals)
            return carry

        lax.fori_loop(0, ept // 64, body, 0)

    @pl.when(c == 0)
    def _():
        run(EPT_C0, s * EPT_C0)

    @pl.when(c == 1)
    def _():
        run(EPT_C1, E_C1_BASE + s * EPT_C1)

    _spmem_tree_reduce(acc_v, sp_sh, red_v, out_v, out_hbm, c, s)


@functools.partial(
    pl.kernel,
    out_type=jax.ShapeDtypeStruct((NC, N_PAD), jnp.float32),
    mesh=_mesh,
    scratch_types=[
        pltpu.VMEM((EPT_C0,), jnp.int32),
        pltpu.VMEM((N_PAD,), jnp.float32),
        pltpu.VMEM((NS, ROWS_PER_TILE), jnp.float32),
        pltpu.VMEM((ROWS_PER_TILE,), jnp.float32),
        pltpu.VMEM_SHARED((NS, N_PAD), jnp.float32),
    ],
    compiler_params=_sc_params,
)
def _sc_deg(dst_hbm, out_hbm, dst_v, acc_v, red_v, out_v, sp_sh):
    c = lax.axis_index("c")
    s = lax.axis_index("s")
    _zero_vmem(acc_v, N_PAD)
    ones = jnp.ones((16,), jnp.float32)

    def run(ept, base):
        pltpu.sync_copy(dst_hbm.at[pl.ds(base, ept)], dst_v.at[pl.ds(0, ept)])

        def body(i, carry):
            for u in range(4):
                dsti = dst_v[pl.ds((i * 4 + u) * 16, 16)]
                plsc.addupdate_scatter(acc_v, [dsti], ones)
            return carry

        lax.fori_loop(0, ept // 64, body, 0)

    @pl.when(c == 0)
    def _():
        run(EPT_C0, s * EPT_C0)

    @pl.when(c == 1)
    def _():
        run(EPT_C1, E_C1_BASE + s * EPT_C1)

    _spmem_tree_reduce(acc_v, sp_sh, red_v, out_v, out_hbm, c, s)


@functools.partial(
    pl.kernel,
    out_type=jax.ShapeDtypeStruct((NC * N_PAD, D_HID), jnp.float32),
    mesh=_mesh,
    scratch_types=[
        pltpu.VMEM((K_C0, CHUNK), jnp.int32),
        pltpu.VMEM((K_C0, CHUNK), jnp.int32),
        pltpu.VMEM((6, CHUNK, D_HID), jnp.float32),
        pltpu.VMEM((ROWS_PER_TILE, D_HID), jnp.float32),
        pltpu.VMEM((5, CHUNK), jnp.int32),
        pltpu.VMEM_SHARED((N_PAD, D_HID), jnp.float32),
    ] + [pltpu.SemaphoreType.DMA] * 12,
    compiler_params=_sc_params_sctile,
)
def _sc_agg_w16(table_hbm, src_hbm, dst_hbm, out_hbm,
                src_v, dst_v, rows_v, obuf_v, oidx_v, acc_sh, *sems):
    NB = 6
    AH = 3
    gsem = sems[:NB]
    ssem = sems[NB:]
    c = lax.axis_index("c")
    s = lax.axis_index("s")
    wid = s * NC + c
    r0 = s * ROWS_PER_TILE
    zv = jnp.zeros((D_HID,), jnp.float32)

    def zbody(r, carry):
        rows_v[0, r] = zv
        return carry

    lax.fori_loop(0, CHUNK, zbody, 0)
    for t in range(ROWS_PER_TILE // CHUNK):
        pltpu.sync_copy(rows_v.at[0],
                        acc_sh.at[pl.ds(r0 + t * CHUNK, CHUNK)])
    plsc.subcore_barrier()

    def issue_gather(j, b):
        pltpu.async_copy(table_hbm.at[src_v.at[j]], rows_v.at[b], gsem[b])

    def wait_gather(j, b):
        pltpu.make_async_copy(
            table_hbm.at[src_v.at[j]], rows_v.at[b], gsem[b]
        ).wait()

    def issue_scatter(j, b):
        pltpu.async_copy(
            rows_v.at[b], acc_sh.at[dst_v.at[j]], ssem[b], add=True
        )

    def wait_scatter(j, b):
        pltpu.make_async_copy(
            rows_v.at[b], acc_sh.at[dst_v.at[j]], ssem[b]
        ).wait()

    def pipe(nchunk, start):
        pltpu.sync_copy(src_hbm.at[pl.ds(start, nchunk)],
                        src_v.at[pl.ds(0, nchunk)])
        pltpu.sync_copy(dst_hbm.at[pl.ds(start, nchunk)],
                        dst_v.at[pl.ds(0, nchunk)])

        for j in range(AH):
            issue_gather(j, j)
        for j in range(AH):
            wait_gather(j, j)
            issue_scatter(j, j)
            issue_gather(j + AH, j + AH)

        def body(j2, carry):
            for u in range(NB):
                j = AH + j2 * NB + u
                b = (AH + u) % NB
                wait_gather(j, b)
                issue_scatter(j, b)
                bw = (AH + u + AH) % NB
                wait_scatter(j - AH, bw)
                issue_gather(j + AH, bw)
            return carry

        steady = nchunk - AH - (NB - 1)
        assert steady % NB == 0
        lax.fori_loop(0, steady // NB, body, 0)

        for j in range(AH + steady, nchunk):
            b = j % NB
            wait_gather(j, b)
            issue_scatter(j, b)
            if j + AH < nchunk:
                bw = (j + AH) % NB
                wait_scatter(j - AH, bw)
                issue_gather(j + AH, bw)
        for j in range(nchunk - NB, nchunk):
            wait_scatter(j, j % NB)

    @pl.when(c == 0)
    def _():
        pipe(K_C0, s * K_C0)

    @pl.when(c == 1)
    def _():
        pipe(K_C1, C1_BASE + s * K_C1)

    plsc.subcore_barrier()

    obase = c * N_PAD + r0
    iota16 = lax.iota(jnp.int32, 16)
    for t in range(ROWS_PER_TILE // CHUNK):
        for u in range(CHUNK // 16):
            oidx_v[t, pl.ds(u * 16, 16)] = obase + t * CHUNK + u * 16 + iota16
    pltpu.sync_copy(acc_sh.at[pl.ds(r0, ROWS_PER_TILE)], obuf_v)
    for t in range(ROWS_PER_TILE // CHUNK):
        pltpu.async_copy(
            obuf_v.at[pl.ds(t * CHUNK, CHUNK)],
            out_hbm.at[oidx_v.at[t]],
            gsem[t % NB],
        )
    for t in range(ROWS_PER_TILE // CHUNK):
        pltpu.make_async_copy(
            obuf_v.at[pl.ds(t * CHUNK, CHUNK)],
            out_hbm.at[oidx_v.at[t]],
            gsem[t % NB],
        ).wait()


BLK = 2048
GRID = N_PAD // BLK


def _tc1_body(x_ref, w_ref, dp_ref, h1s_ref, dis_ref):
    deg = jnp.sum(dp_ref[...], axis=0) + 1.0
    dis = lax.rsqrt(deg)[:, None]
    h = jnp.dot(x_ref[...], w_ref[...], preferred_element_type=jnp.float32)
    h1s_ref[...] = h * dis
    dis_ref[...] = jnp.broadcast_to(dis, (BLK, D_HID))


_tc1 = pl.pallas_call(
    _tc1_body,
    grid=(GRID,),
    in_specs=[
        pl.BlockSpec((BLK, D_IN), lambda i: (i, 0)),
        pl.BlockSpec((D_IN, D_HID), lambda i: (0, 0)),
        pl.BlockSpec((NC, BLK), lambda i: (0, i)),
    ],
    out_specs=[
        pl.BlockSpec((BLK, D_HID), lambda i: (i, 0)),
        pl.BlockSpec((BLK, D_HID), lambda i: (i, 0)),
    ],
    out_shape=[
        jax.ShapeDtypeStruct((N_PAD, D_HID), jnp.float32),
        jax.ShapeDtypeStruct((N_PAD, D_HID), jnp.float32),
    ],
)


def _tc2_body(acc_ref, h1s_ref, dis_ref, b1_ref, w2_ref, h2s_ref):
    tot = acc_ref[0] + acc_ref[1] + h1s_ref[...]
    out1 = tot * dis_ref[...] + b1_ref[...]
    a1 = jnp.maximum(out1, 0.0)
    h2 = jnp.dot(a1, w2_ref[...], preferred_element_type=jnp.float32)
    h2s_ref[...] = h2 * dis_ref[:, :1]


_tc2 = pl.pallas_call(
    _tc2_body,
    grid=(GRID,),
    in_specs=[
        pl.BlockSpec((NC, BLK, D_HID), lambda i: (0, i, 0)),
        pl.BlockSpec((BLK, D_HID), lambda i: (i, 0)),
        pl.BlockSpec((BLK, D_HID), lambda i: (i, 0)),
        pl.BlockSpec((1, D_HID), lambda i: (0, 0)),
        pl.BlockSpec((D_HID, 1), lambda i: (0, 0)),
    ],
    out_specs=pl.BlockSpec((BLK, 1), lambda i: (i, 0)),
    out_shape=jax.ShapeDtypeStruct((N_PAD, 1), jnp.float32),
)


def _tc3_body(a2p_ref, h2s_ref, dis_ref, b2_ref, out_ref):
    tot = jnp.sum(a2p_ref[...], axis=0)[:, None] + h2s_ref[...]
    out_ref[...] = tot * dis_ref[:, :1] + b2_ref[...]


_tc3 = pl.pallas_call(
    _tc3_body,
    grid=(GRID,),
    in_specs=[
        pl.BlockSpec((NC, BLK), lambda i: (0, i)),
        pl.BlockSpec((BLK, 1), lambda i: (i, 0)),
        pl.BlockSpec((BLK, D_HID), lambda i: (i, 0)),
        pl.BlockSpec((1, 1), lambda i: (0, 0)),
    ],
    out_specs=pl.BlockSpec((BLK, 1), lambda i: (i, 0)),
    out_shape=jax.ShapeDtypeStruct((N_PAD, 1), jnp.float32),
)


def kernel(x, edge_index, W1, b1, W2, b2):
    ei = edge_index.astype(jnp.int32)
    pad = jnp.full((E_PAD - E,), TRASH, jnp.int32)
    src = jnp.concatenate([ei[0], pad])
    dst = jnp.concatenate([ei[1], pad])
    src2 = src.reshape(TOT_CHUNKS, CHUNK)
    dst2 = dst.reshape(TOT_CHUNKS, CHUNK)

    x_pad = jnp.pad(x, ((0, N_PAD - N), (0, 0)))
    deg_parts = _sc_deg(dst)
    h1s, dis16 = _tc1(x_pad, W1, deg_parts)
    acc = _sc_agg_w16(h1s, src2, dst2).reshape(NC, N_PAD, D_HID)
    h2s = _tc2(acc, h1s, dis16, b1.reshape(1, D_HID), W2)
    acc2 = _sc_agg_w1(h2s.reshape(-1), src, dst)
    out = _tc3(acc2, h2s, dis16, b2.reshape(1, 1))
    return out[:N]


# speedup vs baseline: 1.0696x; 1.0634x over previous
import functools

import jax
import jax.numpy as jnp
from jax import lax
from jax.experimental import pallas as pl
from jax.experimental.pallas import tpu as pltpu
from jax.experimental.pallas import tpu_sc as plsc

N = 10000
E = 320000
D_IN = 128
D_HID = 16

NC = 2
NS = 16
NW = NC * NS

N_PAD = 10240
TRASH = N
CHUNK = 128
NCHUNK = 80
EPT = NCHUNK * CHUNK
E_PAD = NW * EPT
ROWS_PER_TILE = N_PAD // NS
TOT_CHUNKS = E_PAD // CHUNK
REAL_CHUNKS = E // CHUNK
K_C0 = 128
K_C1 = 32
C1_BASE = NS * K_C0
KPT_C0 = 94
KPT_C1 = 66
S_C1_BASE = NS * KPT_C0
PADC = 60

_mesh = plsc.VectorSubcoreMesh(core_axis_name="c", subcore_axis_name="s")
_sc_params = pltpu.CompilerParams(needs_layout_passes=False)
_sc_params_sctile = pltpu.CompilerParams(
    needs_layout_passes=False, use_tc_tiling_on_sc=False
)


def _zero_vmem(ref, n):
    zv = jnp.zeros((16,), jnp.float32)

    def zb(i, carry):
        ref[pl.ds(i * 16, 16)] = zv
        return carry

    lax.fori_loop(0, n // 16, zb, 0)


def _spmem_tree_reduce(acc_v, sp_sh, red_v, out_v, out_hbm, c, s):
    pltpu.sync_copy(acc_v, sp_sh.at[s])
    plsc.subcore_barrier()
    r0 = s * ROWS_PER_TILE
    for k in range(NS):
        pltpu.sync_copy(sp_sh.at[k].at[pl.ds(r0, ROWS_PER_TILE)], red_v.at[k])

    def rbody(g, carry):
        tot = red_v[0, pl.ds(g * 16, 16)]
        for k in range(1, NS):
            tot = tot + red_v[k, pl.ds(g * 16, 16)]
        out_v[pl.ds(g * 16, 16)] = tot
        return carry

    lax.fori_loop(0, ROWS_PER_TILE // 16, rbody, 0)
    pltpu.sync_copy(out_v, out_hbm.at[c].at[pl.ds(r0, ROWS_PER_TILE)])


def _load_slab(ei3_hbm, padc_hbm, row, dest, base, k_real, k_pad):
    if k_real:
        pltpu.sync_copy(ei3_hbm.at[row].at[pl.ds(base, k_real)],
                        dest.at[pl.ds(0, k_real)])
    if k_pad:
        pltpu.sync_copy(padc_hbm.at[pl.ds(0, k_pad)],
                        dest.at[pl.ds(k_real, k_pad)])


def _scalar_edge_cases(c, s, load_fn):
    @pl.when(c == 0)
    def _():
        load_fn(s * KPT_C0, KPT_C0, 0, KPT_C0)

    @pl.when(jnp.logical_and(c == 1, s < NS - 1))
    def _():
        load_fn(S_C1_BASE + s * KPT_C1, KPT_C1, 0, KPT_C1)

    @pl.when(jnp.logical_and(c == 1, s == NS - 1))
    def _():
        base = S_C1_BASE + (NS - 1) * KPT_C1
        k_real = REAL_CHUNKS - base
        load_fn(base, k_real, KPT_C1 - k_real, KPT_C1)


@functools.partial(
    pl.kernel,
    out_type=jax.ShapeDtypeStruct((NC, N_PAD), jnp.float32),
    mesh=_mesh,
    scratch_types=[
        pltpu.VMEM((KPT_C0, CHUNK), jnp.int32),
        pltpu.VMEM((KPT_C0, CHUNK), jnp.int32),
        pltpu.VMEM((N_PAD,), jnp.float32),
        pltpu.VMEM((N_PAD,), jnp.float32),
        pltpu.VMEM((NS, ROWS_PER_TILE), jnp.float32),
        pltpu.VMEM((ROWS_PER_TILE,), jnp.float32),
        pltpu.VMEM_SHARED((NS, N_PAD), jnp.float32),
    ],
    compiler_params=_sc_params_sctile,
)
def _sc_agg_w1(table_hbm, ei3_hbm, padc_hbm, out_hbm,
               src_v, dst_v, tab_v, acc_v, red_v, out_v, sp_sh):
    c = lax.axis_index("c")
    s = lax.axis_index("s")
    pltpu.sync_copy(table_hbm, tab_v)
    _zero_vmem(acc_v, N_PAD)

    def work(base, k_real, k_pad, kpt):
        _load_slab(ei3_hbm, padc_hbm, 0, src_v, base, k_real, k_pad)
        _load_slab(ei3_hbm, padc_hbm, 1, dst_v, base, k_real, k_pad)

        def body(r, carry):
            for u in range(8):
                srci = src_v[r, pl.ds(u * 16, 16)]
                dsti = dst_v[r, pl.ds(u * 16, 16)]
                vals = plsc.load_gather(tab_v, [srci])
                plsc.addupdate_scatter(acc_v, [dsti], vals)
            return carry

        lax.fori_loop(0, kpt, body, 0)

    _scalar_edge_cases(c, s, work)
    _spmem_tree_reduce(acc_v, sp_sh, red_v, out_v, out_hbm, c, s)


@functools.partial(
    pl.kernel,
    out_type=jax.ShapeDtypeStruct((NC, N_PAD), jnp.float32),
    mesh=_mesh,
    scratch_types=[
        pltpu.VMEM((KPT_C0, CHUNK), jnp.int32),
        pltpu.VMEM((N_PAD,), jnp.float32),
        pltpu.VMEM((NS, ROWS_PER_TILE), jnp.float32),
        pltpu.VMEM((ROWS_PER_TILE,), jnp.float32),
        pltpu.VMEM_SHARED((NS, N_PAD), jnp.float32),
    ],
    compiler_params=_sc_params_sctile,
)
def _sc_deg(ei3_hbm, padc_hbm, out_hbm, dst_v, acc_v, red_v, out_v, sp_sh):
    c = lax.axis_index("c")
    s = lax.axis_index("s")
    _zero_vmem(acc_v, N_PAD)
    ones = jnp.ones((16,), jnp.float32)

    def work(base, k_real, k_pad, kpt):
        _load_slab(ei3_hbm, padc_hbm, 1, dst_v, base, k_real, k_pad)

        def body(r, carry):
            for u in range(8):
                dsti = dst_v[r, pl.ds(u * 16, 16)]
                plsc.addupdate_scatter(acc_v, [dsti], ones)
            return carry

        lax.fori_loop(0, kpt, body, 0)

    _scalar_edge_cases(c, s, work)
    _spmem_tree_reduce(acc_v, sp_sh, red_v, out_v, out_hbm, c, s)


@functools.partial(
    pl.kernel,
    out_type=jax.ShapeDtypeStruct((NC * N_PAD, D_HID), jnp.float32),
    mesh=_mesh,
    scratch_types=[
        pltpu.VMEM((K_C0, CHUNK), jnp.int32),
        pltpu.VMEM((K_C0, CHUNK), jnp.int32),
        pltpu.VMEM((6, CHUNK, D_HID), jnp.float32),
        pltpu.VMEM((ROWS_PER_TILE, D_HID), jnp.float32),
        pltpu.VMEM((5, CHUNK), jnp.int32),
        pltpu.VMEM_SHARED((N_PAD, D_HID), jnp.float32),
    ] + [pltpu.SemaphoreType.DMA] * 12,
    compiler_params=_sc_params_sctile,
)
def _sc_agg_w16(table_hbm, ei3_hbm, padc_hbm, out_hbm,
                src_v, dst_v, rows_v, obuf_v, oidx_v, acc_sh, *sems):
    NB = 6
    AH = 3
    gsem = sems[:NB]
    ssem = sems[NB:]
    c = lax.axis_index("c")
    s = lax.axis_index("s")
    wid = s * NC + c
    r0 = s * ROWS_PER_TILE
    zv = jnp.zeros((D_HID,), jnp.float32)

    def zbody(r, carry):
        rows_v[0, r] = zv
        return carry

    lax.fori_loop(0, CHUNK, zbody, 0)
    for t in range(ROWS_PER_TILE // CHUNK):
        pltpu.sync_copy(rows_v.at[0],
                        acc_sh.at[pl.ds(r0 + t * CHUNK, CHUNK)])
    plsc.subcore_barrier()

    def issue_gather(j, b):
        pltpu.async_copy(table_hbm.at[src_v.at[j]], rows_v.at[b], gsem[b])

    def wait_gather(j, b):
        pltpu.make_async_copy(
            table_hbm.at[src_v.at[j]], rows_v.at[b], gsem[b]
        ).wait()

    def issue_scatter(j, b):
        pltpu.async_copy(
            rows_v.at[b], acc_sh.at[dst_v.at[j]], ssem[b], add=True
        )

    def wait_scatter(j, b):
        pltpu.make_async_copy(
            rows_v.at[b], acc_sh.at[dst_v.at[j]], ssem[b]
        ).wait()

    def pipe(nchunk):
        for j in range(AH):
            issue_gather(j, j)
        for j in range(AH):
            wait_gather(j, j)
            issue_scatter(j, j)
            issue_gather(j + AH, j + AH)

        def body(j2, carry):
            for u in range(NB):
                j = AH + j2 * NB + u
                b = (AH + u) % NB
                wait_gather(j, b)
                issue_scatter(j, b)
                bw = (AH + u + AH) % NB
                wait_scatter(j - AH, bw)
                issue_gather(j + AH, bw)
            return carry

        steady = nchunk - AH - (NB - 1)
        assert steady % NB == 0
        lax.fori_loop(0, steady // NB, body, 0)

        for j in range(AH + steady, nchunk):
            b = j % NB
            wait_gather(j, b)
            issue_scatter(j, b)
            if j + AH < nchunk:
                bw = (j + AH) % NB
                wait_scatter(j - AH, bw)
                issue_gather(j + AH, bw)
        for j in range(nchunk - NB, nchunk):
            wait_scatter(j, j % NB)

    def load(base, k_real, k_pad):
        _load_slab(ei3_hbm, padc_hbm, 0, src_v, base, k_real, k_pad)
        _load_slab(ei3_hbm, padc_hbm, 1, dst_v, base, k_real, k_pad)

    @pl.when(c == 0)
    def _():
        load(s * K_C0, K_C0, 0)
        pipe(K_C0)

    @pl.when(jnp.logical_and(c == 1, s < NS - 2))
    def _():
        load(C1_BASE + s * K_C1, K_C1, 0)

    @pl.when(jnp.logical_and(c == 1, s == NS - 2))
    def _():
        base = C1_BASE + (NS - 2) * K_C1
        load(base, REAL_CHUNKS - base, K_C1 - (REAL_CHUNKS - base))

    @pl.when(jnp.logical_and(c == 1, s == NS - 1))
    def _():
        load(0, 0, K_C1)

    @pl.when(c == 1)
    def _():
        pipe(K_C1)

    plsc.subcore_barrier()

    obase = c * N_PAD + r0
    iota16 = lax.iota(jnp.int32, 16)
    for t in range(ROWS_PER_TILE // CHUNK):
        for u in range(CHUNK // 16):
            oidx_v[t, pl.ds(u * 16, 16)] = obase + t * CHUNK + u * 16 + iota16
    pltpu.sync_copy(acc_sh.at[pl.ds(r0, ROWS_PER_TILE)], obuf_v)
    for t in range(ROWS_PER_TILE // CHUNK):
        pltpu.async_copy(
            obuf_v.at[pl.ds(t * CHUNK, CHUNK)],
            out_hbm.at[oidx_v.at[t]],
            gsem[t % NB],
        )
    for t in range(ROWS_PER_TILE // CHUNK):
        pltpu.make_async_copy(
            obuf_v.at[pl.ds(t * CHUNK, CHUNK)],
            out_hbm.at[oidx_v.at[t]],
            gsem[t % NB],
        ).wait()


BLK = 2048
GRID = N_PAD // BLK


def _tc1_body(x_ref, w_ref, dp_ref, h1s_ref, dis_ref):
    deg = jnp.sum(dp_ref[...], axis=0) + 1.0
    dis = lax.rsqrt(deg)[:, None]
    h = jnp.dot(x_ref[...], w_ref[...], preferred_element_type=jnp.float32)
    h1s_ref[...] = h * dis
    dis_ref[...] = jnp.broadcast_to(dis, (BLK, D_HID))


_tc1 = pl.pallas_call(
    _tc1_body,
    grid=(GRID,),
    in_specs=[
        pl.BlockSpec((BLK, D_IN), lambda i: (i, 0)),
        pl.BlockSpec((D_IN, D_HID), lambda i: (0, 0)),
        pl.BlockSpec((NC, BLK), lambda i: (0, i)),
    ],
    out_specs=[
        pl.BlockSpec((BLK, D_HID), lambda i: (i, 0)),
        pl.BlockSpec((BLK, D_HID), lambda i: (i, 0)),
    ],
    out_shape=[
        jax.ShapeDtypeStruct((N_PAD, D_HID), jnp.float32),
        jax.ShapeDtypeStruct((N_PAD, D_HID), jnp.float32),
    ],
)


def _tc2_body(acc_ref, h1s_ref, dis_ref, b1_ref, w2_ref, h2s_ref):
    tot = acc_ref[0] + acc_ref[1] + h1s_ref[...]
    out1 = tot * dis_ref[...] + b1_ref[...]
    a1 = jnp.maximum(out1, 0.0)
    h2 = jnp.dot(a1, w2_ref[...], preferred_element_type=jnp.float32)
    h2s_ref[...] = h2 * dis_ref[:, :1]


_tc2 = pl.pallas_call(
    _tc2_body,
    grid=(GRID,),
    in_specs=[
        pl.BlockSpec((NC, BLK, D_HID), lambda i: (0, i, 0)),
        pl.BlockSpec((BLK, D_HID), lambda i: (i, 0)),
        pl.BlockSpec((BLK, D_HID), lambda i: (i, 0)),
        pl.BlockSpec((1, D_HID), lambda i: (0, 0)),
        pl.BlockSpec((D_HID, 1), lambda i: (0, 0)),
    ],
    out_specs=pl.BlockSpec((BLK, 1), lambda i: (i, 0)),
    out_shape=jax.ShapeDtypeStruct((N_PAD, 1), jnp.float32),
)


def _tc3_body(a2p_ref, h2s_ref, dis_ref, b2_ref, out_ref):
    tot = jnp.sum(a2p_ref[...], axis=0)[:, None] + h2s_ref[...]
    out_ref[...] = tot * dis_ref[:, :1] + b2_ref[...]


_tc3 = pl.pallas_call(
    _tc3_body,
    grid=(GRID,),
    in_specs=[
        pl.BlockSpec((NC, BLK), lambda i: (0, i)),
        pl.BlockSpec((BLK, 1), lambda i: (i, 0)),
        pl.BlockSpec((BLK, D_HID), lambda i: (i, 0)),
        pl.BlockSpec((1, 1), lambda i: (0, 0)),
    ],
    out_specs=pl.BlockSpec((BLK, 1), lambda i: (i, 0)),
    out_shape=jax.ShapeDtypeStruct((N_PAD, 1), jnp.float32),
)


def kernel(x, edge_index, W1, b1, W2, b2):
    ei3 = edge_index.astype(jnp.int32).reshape(2, REAL_CHUNKS, CHUNK)
    padc = jnp.full((PADC, CHUNK), TRASH, jnp.int32)

    x_pad = jnp.pad(x, ((0, N_PAD - N), (0, 0)))
    deg_parts = _sc_deg(ei3, padc)
    h1s, dis16 = _tc1(x_pad, W1, deg_parts)
    acc = _sc_agg_w16(h1s, ei3, padc).reshape(NC, N_PAD, D_HID)
    h2s = _tc2(acc, h1s, dis16, b1.reshape(1, D_HID), W2)
    acc2 = _sc_agg_w1(h2s.reshape(-1), ei3, padc)
    out = _tc3(acc2, h2s, dis16, b2.reshape(1, 1))
    return out[:N]


# speedup vs baseline: 1.0779x; 1.0077x over previous
import functools

import jax
import jax.numpy as jnp
from jax import lax
from jax.experimental import pallas as pl
from jax.experimental.pallas import tpu as pltpu
from jax.experimental.pallas import tpu_sc as plsc

N = 10000
E = 320000
D_IN = 128
D_HID = 16

NC = 2
NS = 16
NW = NC * NS

N_PAD = 10240
TRASH = N
CHUNK = 128
NCHUNK = 80
EPT = NCHUNK * CHUNK
E_PAD = NW * EPT
ROWS_PER_TILE = N_PAD // NS
TOT_CHUNKS = E_PAD // CHUNK
REAL_CHUNKS = E // CHUNK
K_C0 = 128
K_C1 = 32
C1_BASE = NS * K_C0
KPT_C0 = 94
KPT_C1 = 66
S_C1_BASE = NS * KPT_C0
PADC = 60

_mesh = plsc.VectorSubcoreMesh(core_axis_name="c", subcore_axis_name="s")
_sc_params = pltpu.CompilerParams(needs_layout_passes=False)
_sc_params_sctile = pltpu.CompilerParams(
    needs_layout_passes=False, use_tc_tiling_on_sc=False
)


def _zero_vmem(ref, n):
    zv = jnp.zeros((16,), jnp.float32)

    def zb(i, carry):
        ref[pl.ds(i * 16, 16)] = zv
        return carry

    lax.fori_loop(0, n // 16, zb, 0)


def _spmem_tree_reduce(acc_v, sp_sh, red_v, out_v, out_hbm, c, s):
    pltpu.sync_copy(acc_v, sp_sh.at[s])
    plsc.subcore_barrier()
    r0 = s * ROWS_PER_TILE
    for k in range(NS):
        pltpu.sync_copy(sp_sh.at[k].at[pl.ds(r0, ROWS_PER_TILE)], red_v.at[k])

    def rbody(g, carry):
        tot = red_v[0, pl.ds(g * 16, 16)]
        for k in range(1, NS):
            tot = tot + red_v[k, pl.ds(g * 16, 16)]
        out_v[pl.ds(g * 16, 16)] = tot
        return carry

    lax.fori_loop(0, ROWS_PER_TILE // 16, rbody, 0)
    pltpu.sync_copy(out_v, out_hbm.at[c].at[pl.ds(r0, ROWS_PER_TILE)])


def _load_slab(ei3_hbm, padc_hbm, row, dest, base, k_real, k_pad):
    if k_real:
        pltpu.sync_copy(ei3_hbm.at[row].at[pl.ds(base, k_real)],
                        dest.at[pl.ds(0, k_real)])
    if k_pad:
        pltpu.sync_copy(padc_hbm.at[pl.ds(0, k_pad)],
                        dest.at[pl.ds(k_real, k_pad)])


def _scalar_edge_cases(c, s, load_fn):
    @pl.when(c == 0)
    def _():
        load_fn(s * KPT_C0, KPT_C0, 0, KPT_C0)

    @pl.when(jnp.logical_and(c == 1, s < NS - 1))
    def _():
        load_fn(S_C1_BASE + s * KPT_C1, KPT_C1, 0, KPT_C1)

    @pl.when(jnp.logical_and(c == 1, s == NS - 1))
    def _():
        base = S_C1_BASE + (NS - 1) * KPT_C1
        k_real = REAL_CHUNKS - base
        load_fn(base, k_real, KPT_C1 - k_real, KPT_C1)


@functools.partial(
    pl.kernel,
    out_type=jax.ShapeDtypeStruct((NC, N_PAD), jnp.float32),
    mesh=_mesh,
    scratch_types=[
        pltpu.VMEM((KPT_C0, CHUNK), jnp.int32),
        pltpu.VMEM((KPT_C0, CHUNK), jnp.int32),
        pltpu.VMEM((N_PAD,), jnp.float32),
        pltpu.VMEM((N_PAD,), jnp.float32),
        pltpu.VMEM((NS, ROWS_PER_TILE), jnp.float32),
        pltpu.VMEM((ROWS_PER_TILE,), jnp.float32),
        pltpu.VMEM_SHARED((NS, N_PAD), jnp.float32),
    ],
    compiler_params=_sc_params_sctile,
)
def _sc_agg_w1(table_hbm, ei3_hbm, padc_hbm, out_hbm,
               src_v, dst_v, tab_v, acc_v, red_v, out_v, sp_sh):
    c = lax.axis_index("c")
    s = lax.axis_index("s")
    pltpu.sync_copy(table_hbm, tab_v)
    _zero_vmem(acc_v, N_PAD)

    def work(base, k_real, k_pad, kpt):
        _load_slab(ei3_hbm, padc_hbm, 0, src_v, base, k_real, k_pad)
        _load_slab(ei3_hbm, padc_hbm, 1, dst_v, base, k_real, k_pad)

        def body(r, carry):
            for u in range(8):
                srci = src_v[r, pl.ds(u * 16, 16)]
                dsti = dst_v[r, pl.ds(u * 16, 16)]
                vals = plsc.load_gather(tab_v, [srci])
                plsc.addupdate_scatter(acc_v, [dsti], vals)
            return carry

        lax.fori_loop(0, kpt, body, 0)

    _scalar_edge_cases(c, s, work)
    _spmem_tree_reduce(acc_v, sp_sh, red_v, out_v, out_hbm, c, s)


@functools.partial(
    pl.kernel,
    out_type=jax.ShapeDtypeStruct((NC, N_PAD), jnp.float32),
    mesh=_mesh,
    scratch_types=[
        pltpu.VMEM((KPT_C0, CHUNK), jnp.int32),
        pltpu.VMEM((N_PAD,), jnp.float32),
        pltpu.VMEM((NS, ROWS_PER_TILE), jnp.float32),
        pltpu.VMEM((ROWS_PER_TILE,), jnp.float32),
        pltpu.VMEM_SHARED((NS, N_PAD), jnp.float32),
    ],
    compiler_params=_sc_params_sctile,
)
def _sc_deg(ei3_hbm, padc_hbm, out_hbm, dst_v, acc_v, red_v, out_v, sp_sh):
    c = lax.axis_index("c")
    s = lax.axis_index("s")
    _zero_vmem(acc_v, N_PAD)
    ones = jnp.ones((16,), jnp.float32)

    def work(base, k_real, k_pad, kpt):
        _load_slab(ei3_hbm, padc_hbm, 1, dst_v, base, k_real, k_pad)

        def body(r, carry):
            for u in range(8):
                dsti = dst_v[r, pl.ds(u * 16, 16)]
                plsc.addupdate_scatter(acc_v, [dsti], ones)
            return carry

        lax.fori_loop(0, kpt, body, 0)

    _scalar_edge_cases(c, s, work)
    _spmem_tree_reduce(acc_v, sp_sh, red_v, out_v, out_hbm, c, s)


@functools.partial(
    pl.kernel,
    out_type=jax.ShapeDtypeStruct((NC * N_PAD, D_HID), jnp.float32),
    mesh=_mesh,
    scratch_types=[
        pltpu.VMEM((K_C0, CHUNK), jnp.int32),
        pltpu.VMEM((K_C0, CHUNK), jnp.int32),
        pltpu.VMEM((6, CHUNK, D_HID), jnp.float32),
        pltpu.VMEM((ROWS_PER_TILE, D_HID), jnp.float32),
        pltpu.VMEM((5, CHUNK), jnp.int32),
        pltpu.VMEM_SHARED((N_PAD, D_HID), jnp.float32),
    ] + [pltpu.SemaphoreType.DMA] * 12,
    compiler_params=_sc_params_sctile,
)
def _sc_agg_w16(table_hbm, ei3_hbm, padc_hbm, out_hbm,
                src_v, dst_v, rows_v, obuf_v, oidx_v, acc_sh, *sems):
    NB = 6
    AH = 3
    gsem = sems[:NB]
    ssem = sems[NB:]
    c = lax.axis_index("c")
    s = lax.axis_index("s")
    wid = s * NC + c
    r0 = s * ROWS_PER_TILE
    zv = jnp.zeros((D_HID,), jnp.float32)

    def zbody(r, carry):
        rows_v[0, r] = zv
        return carry

    lax.fori_loop(0, CHUNK, zbody, 0)
    for t in range(ROWS_PER_TILE // CHUNK):
        pltpu.sync_copy(rows_v.at[0],
                        acc_sh.at[pl.ds(r0 + t * CHUNK, CHUNK)])
    plsc.subcore_barrier()

    def issue_gather(j, b):
        pltpu.async_copy(table_hbm.at[src_v.at[j]], rows_v.at[b], gsem[b])

    def wait_gather(j, b):
        pltpu.make_async_copy(
            table_hbm.at[src_v.at[j]], rows_v.at[b], gsem[b]
        ).wait()

    def issue_scatter(j, b):
        pltpu.async_copy(
            rows_v.at[b], acc_sh.at[dst_v.at[j]], ssem[b], add=True
        )

    def wait_scatter(j, b):
        pltpu.make_async_copy(
            rows_v.at[b], acc_sh.at[dst_v.at[j]], ssem[b]
        ).wait()

    def pipe(nchunk):
        for j in range(AH):
            issue_gather(j, j)
        for j in range(AH):
            wait_gather(j, j)
            issue_scatter(j, j)
            issue_gather(j + AH, j + AH)

        def body(j2, carry):
            for u in range(NB):
                j = AH + j2 * NB + u
                b = (AH + u) % NB
                wait_gather(j, b)
                issue_scatter(j, b)
                bw = (AH + u + AH) % NB
                wait_scatter(j - AH, bw)
                issue_gather(j + AH, bw)
            return carry

        steady = nchunk - AH - (NB - 1)
        assert steady % NB == 0
        lax.fori_loop(0, steady // NB, body, 0)

        for j in range(AH + steady, nchunk):
            b = j % NB
            wait_gather(j, b)
            issue_scatter(j, b)
            if j + AH < nchunk:
                bw = (j + AH) % NB
                wait_scatter(j - AH, bw)
                issue_gather(j + AH, bw)
        for j in range(nchunk - NB, nchunk):
            wait_scatter(j, j % NB)

    def load(base, k_real, k_pad):
        _load_slab(ei3_hbm, padc_hbm, 0, src_v, base, k_real, k_pad)
        _load_slab(ei3_hbm, padc_hbm, 1, dst_v, base, k_real, k_pad)

    @pl.when(c == 0)
    def _():
        load(s * K_C0, K_C0, 0)
        pipe(K_C0)

    @pl.when(jnp.logical_and(c == 1, s < NS - 2))
    def _():
        load(C1_BASE + s * K_C1, K_C1, 0)

    @pl.when(jnp.logical_and(c == 1, s == NS - 2))
    def _():
        base = C1_BASE + (NS - 2) * K_C1
        load(base, REAL_CHUNKS - base, K_C1 - (REAL_CHUNKS - base))

    @pl.when(jnp.logical_and(c == 1, s == NS - 1))
    def _():
        load(0, 0, K_C1)

    @pl.when(c == 1)
    def _():
        pipe(K_C1)

    plsc.subcore_barrier()

    obase = c * N_PAD + r0
    iota16 = lax.iota(jnp.int32, 16)
    for t in range(ROWS_PER_TILE // CHUNK):
        for u in range(CHUNK // 16):
            oidx_v[t, pl.ds(u * 16, 16)] = obase + t * CHUNK + u * 16 + iota16
    pltpu.sync_copy(acc_sh.at[pl.ds(r0, ROWS_PER_TILE)], obuf_v)
    for t in range(ROWS_PER_TILE // CHUNK):
        pltpu.async_copy(
            obuf_v.at[pl.ds(t * CHUNK, CHUNK)],
            out_hbm.at[oidx_v.at[t]],
            gsem[t % NB],
        )
    for t in range(ROWS_PER_TILE // CHUNK):
        pltpu.make_async_copy(
            obuf_v.at[pl.ds(t * CHUNK, CHUNK)],
            out_hbm.at[oidx_v.at[t]],
            gsem[t % NB],
        ).wait()


BLK = N_PAD
GRID = N_PAD // BLK


def _tc1_body(x_ref, w_ref, dp_ref, h1s_ref, dis_ref):
    deg = jnp.sum(dp_ref[...], axis=0) + 1.0
    dis = lax.rsqrt(deg)[:, None]
    h = jnp.dot(x_ref[...], w_ref[...], preferred_element_type=jnp.float32)
    h1s_ref[...] = h * dis
    dis_ref[...] = jnp.broadcast_to(dis, (BLK, D_HID))


_tc1 = pl.pallas_call(
    _tc1_body,
    grid=(GRID,),
    in_specs=[
        pl.BlockSpec((BLK, D_IN), lambda i: (i, 0)),
        pl.BlockSpec((D_IN, D_HID), lambda i: (0, 0)),
        pl.BlockSpec((NC, BLK), lambda i: (0, i)),
    ],
    out_specs=[
        pl.BlockSpec((BLK, D_HID), lambda i: (i, 0)),
        pl.BlockSpec((BLK, D_HID), lambda i: (i, 0)),
    ],
    out_shape=[
        jax.ShapeDtypeStruct((N_PAD, D_HID), jnp.float32),
        jax.ShapeDtypeStruct((N_PAD, D_HID), jnp.float32),
    ],
)


def _tc2_body(acc_ref, h1s_ref, dis_ref, b1_ref, w2_ref, h2s_ref):
    tot = acc_ref[0] + acc_ref[1] + h1s_ref[...]
    out1 = tot * dis_ref[...] + b1_ref[...]
    a1 = jnp.maximum(out1, 0.0)
    h2 = jnp.dot(a1, w2_ref[...], preferred_element_type=jnp.float32)
    h2s_ref[...] = h2 * dis_ref[:, :1]


_tc2 = pl.pallas_call(
    _tc2_body,
    grid=(GRID,),
    in_specs=[
        pl.BlockSpec((NC, BLK, D_HID), lambda i: (0, i, 0)),
        pl.BlockSpec((BLK, D_HID), lambda i: (i, 0)),
        pl.BlockSpec((BLK, D_HID), lambda i: (i, 0)),
        pl.BlockSpec((1, D_HID), lambda i: (0, 0)),
        pl.BlockSpec((D_HID, 1), lambda i: (0, 0)),
    ],
    out_specs=pl.BlockSpec((BLK, 1), lambda i: (i, 0)),
    out_shape=jax.ShapeDtypeStruct((N_PAD, 1), jnp.float32),
)


def _tc3_body(a2p_ref, h2s_ref, dis_ref, b2_ref, out_ref):
    tot = jnp.sum(a2p_ref[...], axis=0)[:, None] + h2s_ref[...]
    out_ref[...] = tot * dis_ref[:, :1] + b2_ref[...]


_tc3 = pl.pallas_call(
    _tc3_body,
    grid=(GRID,),
    in_specs=[
        pl.BlockSpec((NC, BLK), lambda i: (0, i)),
        pl.BlockSpec((BLK, 1), lambda i: (i, 0)),
        pl.BlockSpec((BLK, D_HID), lambda i: (i, 0)),
        pl.BlockSpec((1, 1), lambda i: (0, 0)),
    ],
    out_specs=pl.BlockSpec((BLK, 1), lambda i: (i, 0)),
    out_shape=jax.ShapeDtypeStruct((N_PAD, 1), jnp.float32),
)


def kernel(x, edge_index, W1, b1, W2, b2):
    ei3 = edge_index.astype(jnp.int32).reshape(2, REAL_CHUNKS, CHUNK)
    padc = jnp.full((PADC, CHUNK), TRASH, jnp.int32)

    x_pad = jnp.pad(x, ((0, N_PAD - N), (0, 0)))
    deg_parts = _sc_deg(ei3, padc)
    h1s, dis16 = _tc1(x_pad, W1, deg_parts)
    acc = _sc_agg_w16(h1s, ei3, padc).reshape(NC, N_PAD, D_HID)
    h2s = _tc2(acc, h1s, dis16, b1.reshape(1, D_HID), W2)
    acc2 = _sc_agg_w1(h2s.reshape(-1), ei3, padc)
    out = _tc3(acc2, h2s, dis16, b2.reshape(1, 1))
    return out[:N]
